# Initial kernel scaffold; baseline (speedup 1.0000x reference)
import jax
import jax.numpy as jnp
from jax.experimental import pallas as pl


def kernel(x, embed):
    raise NotImplementedError("write your pallas kernel here")


# speedup vs baseline: 1.7550x; 1.7550x over previous
import jax
import jax.numpy as jnp
from jax.experimental import pallas as pl

BN = 1024


def _vq_kernel(x_ref, embed_ref, ind_ref, q_ref):
    x = x_ref[...]
    e = embed_ref[...]
    xe = jax.lax.dot_general(x, e, (((1,), (1,)), ((), ())),
                             preferred_element_type=jnp.float32)
    xsq = jnp.sum(x * x, axis=1, keepdims=True)
    esq = jnp.sum(e * e, axis=1)[None, :]
    dist = -(xsq - 2.0 * xe + esq)
    ind = jnp.argmax(dist, axis=1).astype(jnp.int32)
    ind_ref[0, 0, :] = ind
    k_iota = jax.lax.broadcasted_iota(jnp.int32, (x.shape[0], e.shape[0]), 1)
    onehot = (k_iota == ind[:, None]).astype(jnp.float32)
    q_ref[...] = jax.lax.dot_general(onehot, e, (((1,), (0,)), ((), ())),
                                     preferred_element_type=jnp.float32)


def kernel(x, embed):
    B, T, D = x.shape
    K = embed.shape[0]
    N = B * T
    nb = N // BN
    xf = x.reshape(N, D)
    ind3, q = pl.pallas_call(
        _vq_kernel,
        grid=(nb,),
        in_specs=[pl.BlockSpec((BN, D), lambda i: (i, 0)),
                  pl.BlockSpec((K, D), lambda i: (0, 0))],
        out_specs=[pl.BlockSpec((1, 1, BN), lambda i: (i, 0, 0)),
                   pl.BlockSpec((BN, D), lambda i: (i, 0))],
        out_shape=[jax.ShapeDtypeStruct((nb, 1, BN), jnp.int32),
                   jax.ShapeDtypeStruct((N, D), jnp.float32)],
    )(xf, embed)
    return q.reshape(B, T, D), ind3.reshape(B, T)
